# Initial kernel scaffold; baseline (speedup 1.0000x reference)
import jax
import jax.numpy as jnp
from jax.experimental import pallas as pl


def kernel(student_ids, question_ids_collapsed, labels, ability, difficulty):
    raise NotImplementedError("write your pallas kernel here")


# speedup vs baseline: 1.3492x; 1.3492x over previous
import functools

import jax
import jax.numpy as jnp
from jax import lax
from jax.experimental import pallas as pl
from jax.experimental.pallas import tpu as pltpu
from jax.experimental.pallas import tpu_sc as plsc

_BATCH = 16384
_NC = 2
_NS = 16
_NW = _NC * _NS
_BPW = _BATCH // _NW
_CHUNK = 128
_NCHUNK = _BPW // _CHUNK


def _sc_gather(student_ids, question_ids, ability, difficulty):
    mesh = plsc.VectorSubcoreMesh(core_axis_name="c", subcore_axis_name="s")

    @functools.partial(
        pl.kernel,
        mesh=mesh,
        out_type=(
            jax.ShapeDtypeStruct((_BATCH,), jnp.float32),
            jax.ShapeDtypeStruct((_BATCH,), jnp.float32),
        ),
        scratch_types=[
            pltpu.VMEM((_BPW,), jnp.int32),
            pltpu.VMEM((_BPW,), jnp.int32),
            pltpu.VMEM((_BPW,), jnp.float32),
            pltpu.VMEM((_BPW,), jnp.float32),
            pltpu.SemaphoreType.DMA,
            pltpu.SemaphoreType.DMA,
        ],
    )
    def gather_kernel(sid_hbm, qid_hbm, ab_hbm, df_hbm, a_out, d_out,
                      sidx_v, qidx_v, a_v, d_v, sem_a, sem_d):
        wid = lax.axis_index("s") * _NC + lax.axis_index("c")
        base = wid * _BPW
        pltpu.sync_copy(sid_hbm.at[pl.ds(base, _BPW)], sidx_v)
        pltpu.sync_copy(qid_hbm.at[pl.ds(base, _BPW)], qidx_v)
        copies = []
        for j in range(_NCHUNK):
            sl = pl.ds(j * _CHUNK, _CHUNK)
            copies.append(
                pltpu.async_copy(ab_hbm.at[sidx_v.at[sl]], a_v.at[sl], sem_a))
            copies.append(
                pltpu.async_copy(df_hbm.at[qidx_v.at[sl]], d_v.at[sl], sem_d))
        for c in copies:
            c.wait()
        pltpu.sync_copy(a_v, a_out.at[pl.ds(base, _BPW)])
        pltpu.sync_copy(d_v, d_out.at[pl.ds(base, _BPW)])

    return gather_kernel(student_ids, question_ids, ability, difficulty)


def _tc_finish(a_gathered, d_gathered, labels):
    rows = 128
    cols = _BATCH // rows

    def body(a_ref, d_ref, l_ref, pred_ref, loss_ref):
        sa = jax.nn.softplus(a_ref[...])
        sd = jax.nn.softplus(d_ref[...])
        p = sa - sd
        pred_ref[...] = p
        t = (jnp.maximum(p, 0.0) - p * l_ref[...]
             + jnp.log1p(jnp.exp(-jnp.abs(p))))
        loss_ref[...] = jnp.sum(t).reshape(1, 1) * (1.0 / _BATCH)

    pred, loss = pl.pallas_call(
        body,
        out_shape=(
            jax.ShapeDtypeStruct((rows, cols), jnp.float32),
            jax.ShapeDtypeStruct((1, 1), jnp.float32),
        ),
    )(a_gathered.reshape(rows, cols),
      d_gathered.reshape(rows, cols),
      labels.reshape(rows, cols))
    return loss[0, 0], pred.reshape(_BATCH)


def kernel(student_ids, question_ids_collapsed, labels, ability, difficulty):
    a_vals, d_vals = _sc_gather(student_ids, question_ids_collapsed,
                                ability, difficulty)
    avg_loss, predictions = _tc_finish(a_vals, d_vals, labels)
    return (avg_loss, predictions)
